# Initial kernel scaffold; baseline (speedup 1.0000x reference)
import jax
import jax.numpy as jnp
from jax.experimental import pallas as pl


def kernel(pos, edge_index, edge_attr, batch_index, params):
    raise NotImplementedError("write your pallas kernel here")


# speedup vs baseline: 1.7368x; 1.7368x over previous
import functools

import jax
import jax.numpy as jnp
from jax import lax
from jax.experimental import pallas as pl
from jax.experimental.pallas import tpu as pltpu
from jax.experimental.pallas import tpu_sc as plsc

_N_NODES = 10000
_N_EDGES = 160000
_N_GRAPHS = 16
_HIDDEN = 16
_NW = 32
_EPW = _N_EDGES // _NW
_CH = 100
_CHUNKS = _EPW // _CH
_EBLK = 2000
_NBLK = 2000


def _sc_mesh():
    return plsc.VectorSubcoreMesh(core_axis_name="c", subcore_axis_name="s")


_SC_PARAMS = pltpu.CompilerParams(use_tc_tiling_on_sc=False)


def _sc_gather(x, src2, dst2):
    @functools.partial(
        pl.kernel,
        out_type=[
            jax.ShapeDtypeStruct((_N_EDGES, _HIDDEN), jnp.float32),
            jax.ShapeDtypeStruct((_N_EDGES, _HIDDEN), jnp.float32),
        ],
        mesh=_sc_mesh(),
        compiler_params=_SC_PARAMS,
        scratch_types=[
            pltpu.VMEM((_EPW,), jnp.int32),
            pltpu.VMEM((_EPW, _HIDDEN), jnp.float32),
            pltpu.SemaphoreType.DMA,
        ],
    )
    def k(x_hbm, src_hbm, dst_hbm, xs_hbm, xd_hbm, idx_v, rows_v, sem):
        cid = lax.axis_index("c")
        sid = lax.axis_index("s")
        wid = sid * 2 + cid
        base = wid * _EPW
        for idx_hbm, out_hbm in ((src_hbm, xs_hbm), (dst_hbm, xd_hbm)):
            pltpu.sync_copy(idx_hbm.at[wid], idx_v)
            pltpu.async_copy(x_hbm.at[idx_v], rows_v, sem).wait()
            pltpu.sync_copy(rows_v, out_hbm.at[pl.ds(base, _EPW)])

    return k(x, src2, dst2)


def _sc_scatter(msg, dst3, zeros):
    @functools.partial(
        pl.kernel,
        out_type=jax.ShapeDtypeStruct((2, _N_NODES, _HIDDEN), jnp.float32),
        mesh=_sc_mesh(),
        compiler_params=_SC_PARAMS,
        scratch_types=[
            pltpu.VMEM((_CHUNKS, _CH), jnp.int32),
            pltpu.VMEM((_EPW, _HIDDEN), jnp.float32),
            pltpu.VMEM_SHARED((_N_NODES, _HIDDEN), jnp.float32),
        ],
    )
    def k(msg_hbm, dst_hbm, zero_hbm, out_hbm, idx_v, rows_v, acc_sh):
        cid = lax.axis_index("c")
        sid = lax.axis_index("s")
        wid = sid * 2 + cid
        rps = _N_NODES // 16
        pltpu.sync_copy(zero_hbm.at[pl.ds(sid * rps, rps)],
                        acc_sh.at[pl.ds(sid * rps, rps)])
        pltpu.sync_copy(dst_hbm.at[wid], idx_v)
        pltpu.sync_copy(msg_hbm.at[pl.ds(wid * _EPW, _EPW)], rows_v)
        plsc.subcore_barrier()
        for j in range(_CHUNKS):
            pltpu.sync_copy(rows_v.at[pl.ds(j * _CH, _CH)],
                            acc_sh.at[idx_v.at[j]], add=True)
        plsc.subcore_barrier()
        pltpu.sync_copy(acc_sh.at[pl.ds(sid * rps, rps)],
                        out_hbm.at[cid].at[pl.ds(sid * rps, rps)])

    return k(msg, dst3, zeros)


def _edge_mlp(ea, xs, xd, Wf, bf, Wm, bm, Wl, bl, d_in):
    n_mid = Wm.shape[0]

    def body(ea_ref, xs_ref, xd_ref, Wf_ref, bf_ref, Wm_ref, bm_ref,
             Wl_ref, bl_ref, msg_ref):
        xs_b = xs_ref[...]
        xd_b = xd_ref[...]
        Wf_b = Wf_ref[...]
        h = (jnp.dot(ea_ref[...], Wf_b[0:2], preferred_element_type=jnp.float32)
             + jnp.dot(xs_b[:, :d_in], Wf_b[2:2 + d_in],
                       preferred_element_type=jnp.float32)
             + jnp.dot(xd_b[:, :d_in], Wf_b[2 + d_in:2 + 2 * d_in],
                       preferred_element_type=jnp.float32)
             + bf_ref[...])
        h = jnp.maximum(h, 0.0)
        for i in range(n_mid):
            h = jnp.dot(h, Wm_ref[i], preferred_element_type=jnp.float32) + bm_ref[i]
            h = jnp.maximum(h, 0.0)
        w = jnp.dot(h, Wl_ref[...], preferred_element_type=jnp.float32) + bl_ref[...]
        acc = xs_b[:, 0:1] * w[:, 0:_HIDDEN]
        for i in range(1, d_in):
            acc = acc + xs_b[:, i:i + 1] * w[:, i * _HIDDEN:(i + 1) * _HIDDEN]
        msg_ref[...] = acc

    grid = _N_EDGES // _EBLK
    full = lambda i: (0, 0)
    return pl.pallas_call(
        body,
        grid=(grid,),
        in_specs=[
            pl.BlockSpec((_EBLK, 2), lambda i: (i, 0)),
            pl.BlockSpec((_EBLK, _HIDDEN), lambda i: (i, 0)),
            pl.BlockSpec((_EBLK, _HIDDEN), lambda i: (i, 0)),
            pl.BlockSpec(Wf.shape, full),
            pl.BlockSpec(bf.shape, full),
            pl.BlockSpec(Wm.shape, lambda i: (0, 0, 0)),
            pl.BlockSpec(bm.shape, lambda i: (0, 0, 0)),
            pl.BlockSpec(Wl.shape, full),
            pl.BlockSpec(bl.shape, full),
        ],
        out_specs=pl.BlockSpec((_EBLK, _HIDDEN), lambda i: (i, 0)),
        out_shape=jax.ShapeDtypeStruct((_N_EDGES, _HIDDEN), jnp.float32),
    )(ea, xs, xd, Wf, bf, Wm, bm, Wl, bl)


def _node_update(x, part, rootW, bias, residual):
    def body(x_ref, p_ref, W_ref, b_ref, o_ref):
        xb = x_ref[...]
        out = (jnp.dot(xb, W_ref[...], preferred_element_type=jnp.float32)
               + p_ref[0] + p_ref[1] + b_ref[...])
        if residual:
            out = out + xb
        o_ref[...] = jnp.where(out >= 0.0, out, 0.2 * out)

    grid = _N_NODES // _NBLK
    return pl.pallas_call(
        body,
        grid=(grid,),
        in_specs=[
            pl.BlockSpec((_NBLK, _HIDDEN), lambda i: (i, 0)),
            pl.BlockSpec((2, _NBLK, _HIDDEN), lambda i: (0, i, 0)),
            pl.BlockSpec(rootW.shape, lambda i: (0, 0)),
            pl.BlockSpec(bias.shape, lambda i: (0, 0)),
        ],
        out_specs=pl.BlockSpec((_NBLK, _HIDDEN), lambda i: (i, 0)),
        out_shape=jax.ShapeDtypeStruct((_N_NODES, _HIDDEN), jnp.float32),
    )(x, part, rootW, bias)


def _readout(x, batch2, projW, projb):
    def body(x_ref, b_ref, pW_ref, pb_ref, o_ref, s_acc, c_acc, m_acc):
        i = pl.program_id(0)

        @pl.when(i == 0)
        def _init():
            s_acc[...] = jnp.zeros_like(s_acc)
            c_acc[...] = jnp.zeros_like(c_acc)
            m_acc[...] = jnp.full_like(m_acc, -jnp.inf)

        xb = x_ref[...]
        bi = b_ref[...]
        gids = jax.lax.broadcasted_iota(jnp.int32, (1, _N_GRAPHS), 1)
        onehot = (bi == gids).astype(jnp.float32)
        s_acc[...] += lax.dot_general(onehot, xb, (((0,), (0,)), ((), ())),
                                      preferred_element_type=jnp.float32)
        c_acc[...] += jnp.sum(onehot, axis=0, keepdims=True)
        for g in range(_N_GRAPHS):
            masked = jnp.where(bi == g, xb, -jnp.inf)
            mg = jnp.max(masked, axis=0, keepdims=True)
            m_acc[g:g + 1, :] = jnp.maximum(m_acc[g:g + 1, :], mg)

        @pl.when(i == pl.num_programs(0) - 1)
        def _fin():
            s = s_acc[...]
            cnt = jnp.maximum(c_acc[...], 1.0)
            mean = s / cnt.reshape(_N_GRAPHS, 1)
            cat = jnp.concatenate([s, mean, m_acc[...]], axis=1)
            o_ref[...] = (jnp.dot(cat, pW_ref[...],
                                  preferred_element_type=jnp.float32)
                          + pb_ref[...])

    grid = _N_NODES // _NBLK
    return pl.pallas_call(
        body,
        grid=(grid,),
        in_specs=[
            pl.BlockSpec((_NBLK, _HIDDEN), lambda i: (i, 0)),
            pl.BlockSpec((_NBLK, 1), lambda i: (i, 0)),
            pl.BlockSpec(projW.shape, lambda i: (0, 0)),
            pl.BlockSpec(projb.shape, lambda i: (0, 0)),
        ],
        out_specs=pl.BlockSpec((_N_GRAPHS, 1), lambda i: (0, 0)),
        out_shape=jax.ShapeDtypeStruct((_N_GRAPHS, 1), jnp.float32),
        scratch_shapes=[
            pltpu.VMEM((_N_GRAPHS, _HIDDEN), jnp.float32),
            pltpu.VMEM((1, _N_GRAPHS), jnp.float32),
            pltpu.VMEM((_N_GRAPHS, _HIDDEN), jnp.float32),
        ],
    )(x, batch2, projW, projb)


def kernel(pos, edge_index, edge_attr, batch_index, params):
    src = edge_index[0].astype(jnp.int32)
    dst = edge_index[1].astype(jnp.int32)
    src2 = src.reshape(_NW, _EPW)
    dst2 = dst.reshape(_NW, _EPW)
    dst3 = dst.reshape(_NW, _CHUNKS, _CH)
    ea = edge_attr.astype(jnp.float32)
    zeros = jnp.zeros((_N_NODES, _HIDDEN), jnp.float32)

    x = jnp.pad(pos.astype(jnp.float32), ((0, 0), (0, _HIDDEN - pos.shape[1])))

    for l, layer in enumerate(params["layers"]):
        d_in = 2 if l == 0 else _HIDDEN
        mlp = layer["edge_mlp"]
        Wf, bf = mlp[0]
        Wm = jnp.stack([W for W, _ in mlp[1:15]])
        bm = jnp.stack([b.reshape(1, -1) for _, b in mlp[1:15]])
        Wl, bl = mlp[15]
        bf = bf.reshape(1, -1)
        bl = bl.reshape(1, -1)
        rootW = layer["root_W"]
        if rootW.shape[0] < _HIDDEN:
            rootW = jnp.pad(rootW, ((0, _HIDDEN - rootW.shape[0]), (0, 0)))
        bias = layer["bias"].reshape(1, -1)

        xs, xd = _sc_gather(x, src2, dst2)
        msg = _edge_mlp(ea, xs, xd, Wf, bf, Wm, bm, Wl, bl, d_in)
        part = _sc_scatter(msg, dst3, zeros)
        x = _node_update(x, part, rootW, bias, residual=(l > 0))

    out = _readout(x, batch_index.astype(jnp.int32).reshape(_N_NODES, 1),
                   params["proj_W"], params["proj_b"].reshape(1, 1))
    return out.reshape(_N_GRAPHS)


# speedup vs baseline: 3.7689x; 2.1700x over previous
import functools

import jax
import jax.numpy as jnp
from jax import lax
from jax.experimental import pallas as pl
from jax.experimental.pallas import tpu as pltpu
from jax.experimental.pallas import tpu_sc as plsc

_N_NODES = 10000
_N_EDGES = 160000
_N_GRAPHS = 16
_HIDDEN = 16
_NW = 32
_EPW = _N_EDGES // _NW
_CH = 100
_CHUNKS = _EPW // _CH
_EBLK = 3200
_EROWS = _EBLK // 8
_NROWS = _N_NODES // 8
_NBLK = 250


def _sc_mesh():
    return plsc.VectorSubcoreMesh(core_axis_name="c", subcore_axis_name="s")


_SC_PARAMS = pltpu.CompilerParams(use_tc_tiling_on_sc=False)
_F32 = jnp.float32


def _sc_gather(x, src2, dst2):
    @functools.partial(
        pl.kernel,
        out_type=[
            jax.ShapeDtypeStruct((_N_EDGES, _HIDDEN), _F32),
            jax.ShapeDtypeStruct((_N_EDGES, _HIDDEN), _F32),
        ],
        mesh=_sc_mesh(),
        compiler_params=_SC_PARAMS,
        scratch_types=[
            pltpu.VMEM((_EPW,), jnp.int32),
            pltpu.VMEM((_EPW, _HIDDEN), _F32),
            pltpu.SemaphoreType.DMA,
        ],
    )
    def k(x_hbm, src_hbm, dst_hbm, xs_hbm, xd_hbm, idx_v, rows_v, sem):
        cid = lax.axis_index("c")
        sid = lax.axis_index("s")
        wid = sid * 2 + cid
        base = wid * _EPW
        for idx_hbm, out_hbm in ((src_hbm, xs_hbm), (dst_hbm, xd_hbm)):
            pltpu.sync_copy(idx_hbm.at[wid], idx_v)
            pltpu.async_copy(x_hbm.at[idx_v], rows_v, sem).wait()
            pltpu.sync_copy(rows_v, out_hbm.at[pl.ds(base, _EPW)])

    return k(x, src2, dst2)


def _sc_scatter(msg, dst3, zeros):
    @functools.partial(
        pl.kernel,
        out_type=jax.ShapeDtypeStruct((2, _N_NODES, _HIDDEN), _F32),
        mesh=_sc_mesh(),
        compiler_params=_SC_PARAMS,
        scratch_types=[
            pltpu.VMEM((_CHUNKS, _CH), jnp.int32),
            pltpu.VMEM((_EPW, _HIDDEN), _F32),
            pltpu.VMEM_SHARED((_N_NODES, _HIDDEN), _F32),
        ],
    )
    def k(msg_hbm, dst_hbm, zero_hbm, out_hbm, idx_v, rows_v, acc_sh):
        cid = lax.axis_index("c")
        sid = lax.axis_index("s")
        wid = sid * 2 + cid
        rps = _N_NODES // 16
        pltpu.sync_copy(zero_hbm.at[pl.ds(sid * rps, rps)],
                        acc_sh.at[pl.ds(sid * rps, rps)])
        pltpu.sync_copy(dst_hbm.at[wid], idx_v)
        pltpu.sync_copy(msg_hbm.at[pl.ds(wid * _EPW, _EPW)], rows_v)
        plsc.subcore_barrier()
        for j in range(_CHUNKS):
            pltpu.sync_copy(rows_v.at[pl.ds(j * _CH, _CH)],
                            acc_sh.at[idx_v.at[j]], add=True)
        plsc.subcore_barrier()
        pltpu.sync_copy(acc_sh.at[pl.ds(sid * rps, rps)],
                        out_hbm.at[cid].at[pl.ds(sid * rps, rps)])

    return k(msg, dst3, zeros)


def _edge_mlp(eaP, xsP, xdP, Wea, Wxs, Wxd, bfP, WmP, bmP, WlP, blP, RP, SP):
    n_mid = WmP.shape[0]

    def body(ea_ref, xs_ref, xd_ref, Wea_ref, Wxs_ref, Wxd_ref, bf_ref,
             WmP_ref, bmP_ref, WlP_ref, blP_ref, RP_ref, SP_ref, msg_ref):
        xsb = xs_ref[...]
        h = (jnp.dot(ea_ref[...], Wea_ref[...], preferred_element_type=_F32)
             + jnp.dot(xsb, Wxs_ref[...], preferred_element_type=_F32)
             + jnp.dot(xd_ref[...], Wxd_ref[...], preferred_element_type=_F32)
             + bf_ref[...])
        h = jnp.maximum(h, 0.0)
        for s in range(n_mid):
            h = jnp.dot(h, WmP_ref[s], preferred_element_type=_F32) + bmP_ref[s]
            h = jnp.maximum(h, 0.0)
        w = jnp.dot(h, WlP_ref[...], preferred_element_type=_F32) + blP_ref[...]
        xe = jnp.dot(xsb, RP_ref[...], preferred_element_type=_F32)
        msg_ref[...] = jnp.dot(w * xe, SP_ref[...], preferred_element_type=_F32)

    grid = _N_EDGES // _EBLK
    full2 = lambda i: (0, 0)
    blk = lambda i: (i, 0)
    return pl.pallas_call(
        body,
        grid=(grid,),
        in_specs=[
            pl.BlockSpec((_EROWS, 128), blk),
            pl.BlockSpec((_EROWS, 128), blk),
            pl.BlockSpec((_EROWS, 128), blk),
            pl.BlockSpec(Wea.shape, full2),
            pl.BlockSpec(Wxs.shape, full2),
            pl.BlockSpec(Wxd.shape, full2),
            pl.BlockSpec(bfP.shape, full2),
            pl.BlockSpec(WmP.shape, lambda i: (0, 0, 0)),
            pl.BlockSpec(bmP.shape, lambda i: (0, 0, 0)),
            pl.BlockSpec(WlP.shape, full2),
            pl.BlockSpec(blP.shape, full2),
            pl.BlockSpec(RP.shape, full2),
            pl.BlockSpec(SP.shape, full2),
        ],
        out_specs=pl.BlockSpec((_EROWS, 128), blk),
        out_shape=jax.ShapeDtypeStruct((_N_EDGES // 8, 128), _F32),
    )(eaP, xsP, xdP, Wea, Wxs, Wxd, bfP, WmP, bmP, WlP, blP, RP, SP)


def _node_update(xP, partP, rWP, biasP, residual):
    def body(x_ref, p_ref, W_ref, b_ref, o_ref):
        xb = x_ref[...]
        out = (jnp.dot(xb, W_ref[...], preferred_element_type=_F32)
               + p_ref[0] + p_ref[1] + b_ref[...])
        if residual:
            out = out + xb
        o_ref[...] = jnp.maximum(out, 0.2 * out)

    return pl.pallas_call(
        body,
        grid=(1,),
        in_specs=[
            pl.BlockSpec((_NROWS, 128), lambda i: (0, 0)),
            pl.BlockSpec((2, _NROWS, 128), lambda i: (0, 0, 0)),
            pl.BlockSpec(rWP.shape, lambda i: (0, 0)),
            pl.BlockSpec(biasP.shape, lambda i: (0, 0)),
        ],
        out_specs=pl.BlockSpec((_NROWS, 128), lambda i: (0, 0)),
        out_shape=jax.ShapeDtypeStruct((_NROWS, 128), _F32),
    )(xP, partP, rWP, biasP)


def _readout(x, batch2, projW, projb):
    def body(x_ref, b_ref, pW_ref, pb_ref, o_ref, s_acc, c_acc, m_acc):
        i = pl.program_id(0)

        @pl.when(i == 0)
        def _init():
            s_acc[...] = jnp.zeros_like(s_acc)
            c_acc[...] = jnp.zeros_like(c_acc)
            m_acc[...] = jnp.full_like(m_acc, -jnp.inf)

        xb = x_ref[...]
        bi = b_ref[...]
        gids = lax.broadcasted_iota(jnp.int32, (1, _N_GRAPHS), 1)
        onehot = (bi == gids).astype(_F32)
        s_acc[...] += lax.dot_general(onehot, xb, (((0,), (0,)), ((), ())),
                                      preferred_element_type=_F32)
        c_acc[...] += jnp.sum(onehot, axis=0, keepdims=True)
        for g in range(_N_GRAPHS):
            masked = jnp.where(bi == g, xb, -jnp.inf)
            mg = jnp.max(masked, axis=0, keepdims=True)
            m_acc[g:g + 1, :] = jnp.maximum(m_acc[g:g + 1, :], mg)

        @pl.when(i == pl.num_programs(0) - 1)
        def _fin():
            s = s_acc[...]
            cnt = jnp.maximum(c_acc[...], 1.0)
            mean = s / cnt.reshape(_N_GRAPHS, 1)
            cat = jnp.concatenate([s, mean, m_acc[...]], axis=1)
            o_ref[...] = (jnp.dot(cat, pW_ref[...],
                                  preferred_element_type=_F32) + pb_ref[...])

    nblk = 2000
    grid = _N_NODES // nblk
    return pl.pallas_call(
        body,
        grid=(grid,),
        in_specs=[
            pl.BlockSpec((nblk, _HIDDEN), lambda i: (i, 0)),
            pl.BlockSpec((nblk, 1), lambda i: (i, 0)),
            pl.BlockSpec(projW.shape, lambda i: (0, 0)),
            pl.BlockSpec(projb.shape, lambda i: (0, 0)),
        ],
        out_specs=pl.BlockSpec((_N_GRAPHS, 1), lambda i: (0, 0)),
        out_shape=jax.ShapeDtypeStruct((_N_GRAPHS, 1), _F32),
        scratch_shapes=[
            pltpu.VMEM((_N_GRAPHS, _HIDDEN), _F32),
            pltpu.VMEM((1, _N_GRAPHS), _F32),
            pltpu.VMEM((_N_GRAPHS, _HIDDEN), _F32),
        ],
    )(x, batch2, projW, projb)


def kernel(pos, edge_index, edge_attr, batch_index, params):
    src = edge_index[0].astype(jnp.int32)
    dst = edge_index[1].astype(jnp.int32)
    src2 = src.reshape(_NW, _EPW)
    dst2 = dst.reshape(_NW, _EPW)
    dst3 = dst.reshape(_NW, _CHUNKS, _CH)
    zeros = jnp.zeros((_N_NODES, _HIDDEN), _F32)
    eye8 = jnp.eye(8, dtype=_F32)

    eaP = jnp.pad(edge_attr.astype(_F32), ((0, 0), (0, _HIDDEN - 2))
                  ).reshape(_N_EDGES // 8, 128)
    xP = jnp.pad(pos.astype(_F32), ((0, 0), (0, _HIDDEN - pos.shape[1]))
                 ).reshape(_NROWS, 128)

    for l, layer in enumerate(params["layers"]):
        d_in = 2 if l == 0 else _HIDDEN
        dm = d_in * _HIDDEN
        mlp = layer["edge_mlp"]
        Wf, bf = mlp[0]
        Wea = jnp.kron(eye8, jnp.pad(Wf[0:2], ((0, 14), (0, 0))))
        Wxs = jnp.kron(eye8, jnp.pad(Wf[2:2 + d_in],
                                     ((0, _HIDDEN - d_in), (0, 0))))
        Wxd = jnp.kron(eye8, jnp.pad(Wf[2 + d_in:2 + 2 * d_in],
                                     ((0, _HIDDEN - d_in), (0, 0))))
        bfP = jnp.tile(bf.reshape(1, -1), (1, 8))
        WmP = jnp.stack([jnp.kron(eye8, W) for W, _ in mlp[1:15]])
        bmP = jnp.stack([jnp.tile(b.reshape(1, -1), (1, 8))
                         for _, b in mlp[1:15]])
        Wl, bl = mlp[15]
        WlP = jnp.kron(eye8, Wl)
        blP = jnp.tile(bl.reshape(1, -1), (1, 8))
        ri = lax.broadcasted_iota(jnp.int32, (_HIDDEN, dm), 0)
        ci = lax.broadcasted_iota(jnp.int32, (_HIDDEN, dm), 1)
        R = (ci // _HIDDEN == ri).astype(_F32)
        ki = lax.broadcasted_iota(jnp.int32, (dm, _HIDDEN), 0)
        oi = lax.broadcasted_iota(jnp.int32, (dm, _HIDDEN), 1)
        S = (ki % _HIDDEN == oi).astype(_F32)
        RP = jnp.kron(eye8, R)
        SP = jnp.kron(eye8, S)
        rootW = layer["root_W"]
        if rootW.shape[0] < _HIDDEN:
            rootW = jnp.pad(rootW, ((0, _HIDDEN - rootW.shape[0]), (0, 0)))
        rWP = jnp.kron(eye8, rootW)
        biasP = jnp.tile(layer["bias"].reshape(1, -1), (1, 8))

        x16 = xP.reshape(_N_NODES, _HIDDEN)
        xs, xd = _sc_gather(x16, src2, dst2)
        xsP = xs.reshape(_N_EDGES // 8, 128)
        xdP = xd.reshape(_N_EDGES // 8, 128)
        msgP = _edge_mlp(eaP, xsP, xdP, Wea, Wxs, Wxd, bfP, WmP, bmP,
                         WlP, blP, RP, SP)
        part = _sc_scatter(msgP.reshape(_N_EDGES, _HIDDEN), dst3, zeros)
        partP = part.reshape(2, _NROWS, 128)
        xP = _node_update(xP, partP, rWP, biasP, residual=(l > 0))

    out = _readout(xP.reshape(_N_NODES, _HIDDEN),
                   batch_index.astype(jnp.int32).reshape(_N_NODES, 1),
                   params["proj_W"], params["proj_b"].reshape(1, 1))
    return out.reshape(_N_GRAPHS)


# speedup vs baseline: 4.5558x; 1.2088x over previous
import functools

import jax
import jax.numpy as jnp
from jax import lax
from jax.experimental import pallas as pl
from jax.experimental.pallas import tpu as pltpu
from jax.experimental.pallas import tpu_sc as plsc

_N_NODES = 10000
_N_EDGES = 160000
_N_GRAPHS = 16
_HIDDEN = 16
_NW = 32
_EPW = _N_EDGES // _NW
_CH = 100
_CHUNKS = _EPW // _CH
_EBLK = 6400
_EROWS = _EBLK // 8
_NROWS = _N_NODES // 8
_NBLK = 250


def _sc_mesh():
    return plsc.VectorSubcoreMesh(core_axis_name="c", subcore_axis_name="s")


_SC_PARAMS = pltpu.CompilerParams(use_tc_tiling_on_sc=False)
_F32 = jnp.float32


def _sc_gather(x, src2, dst2):
    @functools.partial(
        pl.kernel,
        out_type=[
            jax.ShapeDtypeStruct((_N_EDGES, _HIDDEN), _F32),
            jax.ShapeDtypeStruct((_N_EDGES, _HIDDEN), _F32),
        ],
        mesh=_sc_mesh(),
        compiler_params=_SC_PARAMS,
        scratch_types=[
            pltpu.VMEM((_EPW,), jnp.int32),
            pltpu.VMEM((_EPW, _HIDDEN), _F32),
            pltpu.SemaphoreType.DMA,
        ],
    )
    def k(x_hbm, src_hbm, dst_hbm, xs_hbm, xd_hbm, idx_v, rows_v, sem):
        cid = lax.axis_index("c")
        sid = lax.axis_index("s")
        wid = sid * 2 + cid
        base = wid * _EPW
        for idx_hbm, out_hbm in ((src_hbm, xs_hbm), (dst_hbm, xd_hbm)):
            pltpu.sync_copy(idx_hbm.at[wid], idx_v)
            pltpu.async_copy(x_hbm.at[idx_v], rows_v, sem).wait()
            pltpu.sync_copy(rows_v, out_hbm.at[pl.ds(base, _EPW)])

    return k(x, src2, dst2)


def _sc_scatter(msg, dst3, zeros):
    @functools.partial(
        pl.kernel,
        out_type=jax.ShapeDtypeStruct((2, _N_NODES, _HIDDEN), _F32),
        mesh=_sc_mesh(),
        compiler_params=_SC_PARAMS,
        scratch_types=[
            pltpu.VMEM((_CHUNKS, _CH), jnp.int32),
            pltpu.VMEM((_EPW, _HIDDEN), _F32),
            pltpu.VMEM_SHARED((_N_NODES, _HIDDEN), _F32),
        ],
    )
    def k(msg_hbm, dst_hbm, zero_hbm, out_hbm, idx_v, rows_v, acc_sh):
        cid = lax.axis_index("c")
        sid = lax.axis_index("s")
        wid = sid * 2 + cid
        rps = _N_NODES // 16
        pltpu.sync_copy(zero_hbm.at[pl.ds(sid * rps, rps)],
                        acc_sh.at[pl.ds(sid * rps, rps)])
        pltpu.sync_copy(dst_hbm.at[wid], idx_v)
        pltpu.sync_copy(msg_hbm.at[pl.ds(wid * _EPW, _EPW)], rows_v)
        plsc.subcore_barrier()
        for j in range(_CHUNKS):
            pltpu.sync_copy(rows_v.at[pl.ds(j * _CH, _CH)],
                            acc_sh.at[idx_v.at[j]], add=True)
        plsc.subcore_barrier()
        pltpu.sync_copy(acc_sh.at[pl.ds(sid * rps, rps)],
                        out_hbm.at[cid].at[pl.ds(sid * rps, rps)])

    return k(msg, dst3, zeros)


def _edge_mlp(eaP, xsP, xdP, Wf, bf, Wm, bm, Wl, bl, d_in):
    n_mid = Wm.shape[0]
    dm = d_in * _HIDDEN
    i32 = jnp.int32

    def body(ea_ref, xs_ref, xd_ref, Wf_ref, bf_ref, Wm_ref, bm_ref,
             Wl_ref, bl_ref, msg_ref,
             Wea_s, Wxs_s, Wxd_s, WmP_s, WlP_s, RP_s, SP_s):

        def expand(W, a, b):
            r = W.shape[0]
            ra = lax.broadcasted_iota(i32, (8 * a, r), 0)
            ca = lax.broadcasted_iota(i32, (8 * a, r), 1)
            A = ((ra % a) == ca).astype(_F32)
            rb = lax.broadcasted_iota(i32, (b, 8 * b), 0)
            cb = lax.broadcasted_iota(i32, (b, 8 * b), 1)
            B = (rb == (cb % b)).astype(_F32)
            T = jnp.dot(jnp.dot(A, W, preferred_element_type=_F32), B,
                        preferred_element_type=_F32)
            rm = lax.broadcasted_iota(i32, (8 * a, 8 * b), 0)
            cm = lax.broadcasted_iota(i32, (8 * a, 8 * b), 1)
            return jnp.where((rm // a) == (cm // b), T, 0.0)

        @pl.when(pl.program_id(0) == 0)
        def _build():
            Wf_b = Wf_ref[...]
            Wea_s[...] = expand(Wf_b[0:2], _HIDDEN, 64)
            Wxs_s[...] = expand(Wf_b[2:2 + d_in], _HIDDEN, 64)
            Wxd_s[...] = expand(Wf_b[2 + d_in:2 + 2 * d_in], _HIDDEN, 64)
            for s in range(n_mid):
                WmP_s[s] = expand(Wm_ref[s], 64, 64)
            WlP_s[...] = expand(Wl_ref[...], 64, dm)
            ri = lax.broadcasted_iota(i32, (128, 8 * dm), 0)
            ci = lax.broadcasted_iota(i32, (128, 8 * dm), 1)
            RP_s[...] = ((ci // dm == ri // _HIDDEN)
                         & ((ci % dm) // _HIDDEN == ri % _HIDDEN)).astype(_F32)
            ki = lax.broadcasted_iota(i32, (8 * dm, 128), 0)
            oi = lax.broadcasted_iota(i32, (8 * dm, 128), 1)
            SP_s[...] = ((ki // dm == oi // _HIDDEN)
                         & (ki % _HIDDEN == oi % _HIDDEN)).astype(_F32)

        xsb = xs_ref[...]
        h = (jnp.dot(ea_ref[...], Wea_s[...], preferred_element_type=_F32)
             + jnp.dot(xsb, Wxs_s[...], preferred_element_type=_F32)
             + jnp.dot(xd_ref[...], Wxd_s[...], preferred_element_type=_F32)
             + jnp.tile(bf_ref[...], (1, 8)))
        h = jnp.maximum(h, 0.0)
        for s in range(n_mid):
            h = (jnp.dot(h, WmP_s[s], preferred_element_type=_F32)
                 + jnp.tile(bm_ref[s], (1, 8)))
            h = jnp.maximum(h, 0.0)
        w = (jnp.dot(h, WlP_s[...], preferred_element_type=_F32)
             + jnp.tile(bl_ref[...], (1, 8)))
        xe = jnp.dot(xsb, RP_s[...], preferred_element_type=_F32)
        msg_ref[...] = jnp.dot(w * xe, SP_s[...], preferred_element_type=_F32)

    grid = _N_EDGES // _EBLK
    full2 = lambda i: (0, 0)
    blk = lambda i: (i, 0)
    return pl.pallas_call(
        body,
        grid=(grid,),
        in_specs=[
            pl.BlockSpec((_EROWS, 128), blk),
            pl.BlockSpec((_EROWS, 128), blk),
            pl.BlockSpec((_EROWS, 128), blk),
            pl.BlockSpec(Wf.shape, full2),
            pl.BlockSpec(bf.shape, full2),
            pl.BlockSpec(Wm.shape, lambda i: (0, 0, 0)),
            pl.BlockSpec(bm.shape, lambda i: (0, 0, 0)),
            pl.BlockSpec(Wl.shape, full2),
            pl.BlockSpec(bl.shape, full2),
        ],
        out_specs=pl.BlockSpec((_EROWS, 128), blk),
        out_shape=jax.ShapeDtypeStruct((_N_EDGES // 8, 128), _F32),
        scratch_shapes=[
            pltpu.VMEM((128, 512), _F32),
            pltpu.VMEM((128, 512), _F32),
            pltpu.VMEM((128, 512), _F32),
            pltpu.VMEM((n_mid, 512, 512), _F32),
            pltpu.VMEM((512, 8 * dm), _F32),
            pltpu.VMEM((128, 8 * dm), _F32),
            pltpu.VMEM((8 * dm, 128), _F32),
        ],
    )(eaP, xsP, xdP, Wf, bf, Wm, bm, Wl, bl)


def _node_update(xP, partP, rWP, biasP, residual):
    def body(x_ref, p_ref, W_ref, b_ref, o_ref):
        xb = x_ref[...]
        out = (jnp.dot(xb, W_ref[...], preferred_element_type=_F32)
               + p_ref[0] + p_ref[1] + b_ref[...])
        if residual:
            out = out + xb
        o_ref[...] = jnp.maximum(out, 0.2 * out)

    return pl.pallas_call(
        body,
        grid=(1,),
        in_specs=[
            pl.BlockSpec((_NROWS, 128), lambda i: (0, 0)),
            pl.BlockSpec((2, _NROWS, 128), lambda i: (0, 0, 0)),
            pl.BlockSpec(rWP.shape, lambda i: (0, 0)),
            pl.BlockSpec(biasP.shape, lambda i: (0, 0)),
        ],
        out_specs=pl.BlockSpec((_NROWS, 128), lambda i: (0, 0)),
        out_shape=jax.ShapeDtypeStruct((_NROWS, 128), _F32),
    )(xP, partP, rWP, biasP)


def _readout(x, batch2, projW, projb):
    def body(x_ref, b_ref, pW_ref, pb_ref, o_ref, s_acc, c_acc, m_acc):
        i = pl.program_id(0)

        @pl.when(i == 0)
        def _init():
            s_acc[...] = jnp.zeros_like(s_acc)
            c_acc[...] = jnp.zeros_like(c_acc)
            m_acc[...] = jnp.full_like(m_acc, -jnp.inf)

        xb = x_ref[...]
        bi = b_ref[...]
        gids = lax.broadcasted_iota(jnp.int32, (1, _N_GRAPHS), 1)
        onehot = (bi == gids).astype(_F32)
        s_acc[...] += lax.dot_general(onehot, xb, (((0,), (0,)), ((), ())),
                                      preferred_element_type=_F32)
        c_acc[...] += jnp.sum(onehot, axis=0, keepdims=True)
        for g in range(_N_GRAPHS):
            masked = jnp.where(bi == g, xb, -jnp.inf)
            mg = jnp.max(masked, axis=0, keepdims=True)
            m_acc[g:g + 1, :] = jnp.maximum(m_acc[g:g + 1, :], mg)

        @pl.when(i == pl.num_programs(0) - 1)
        def _fin():
            s = s_acc[...]
            cnt = jnp.maximum(c_acc[...], 1.0)
            mean = s / cnt.reshape(_N_GRAPHS, 1)
            cat = jnp.concatenate([s, mean, m_acc[...]], axis=1)
            o_ref[...] = (jnp.dot(cat, pW_ref[...],
                                  preferred_element_type=_F32) + pb_ref[...])

    nblk = 2000
    grid = _N_NODES // nblk
    return pl.pallas_call(
        body,
        grid=(grid,),
        in_specs=[
            pl.BlockSpec((nblk, _HIDDEN), lambda i: (i, 0)),
            pl.BlockSpec((nblk, 1), lambda i: (i, 0)),
            pl.BlockSpec(projW.shape, lambda i: (0, 0)),
            pl.BlockSpec(projb.shape, lambda i: (0, 0)),
        ],
        out_specs=pl.BlockSpec((_N_GRAPHS, 1), lambda i: (0, 0)),
        out_shape=jax.ShapeDtypeStruct((_N_GRAPHS, 1), _F32),
        scratch_shapes=[
            pltpu.VMEM((_N_GRAPHS, _HIDDEN), _F32),
            pltpu.VMEM((1, _N_GRAPHS), _F32),
            pltpu.VMEM((_N_GRAPHS, _HIDDEN), _F32),
        ],
    )(x, batch2, projW, projb)


def kernel(pos, edge_index, edge_attr, batch_index, params):
    src = edge_index[0].astype(jnp.int32)
    dst = edge_index[1].astype(jnp.int32)
    src2 = src.reshape(_NW, _EPW)
    dst2 = dst.reshape(_NW, _EPW)
    dst3 = dst.reshape(_NW, _CHUNKS, _CH)
    zeros = jnp.zeros((_N_NODES, _HIDDEN), _F32)
    eye8 = jnp.eye(8, dtype=_F32)

    eaP = jnp.pad(edge_attr.astype(_F32), ((0, 0), (0, _HIDDEN - 2))
                  ).reshape(_N_EDGES // 8, 128)
    xP = jnp.pad(pos.astype(_F32), ((0, 0), (0, _HIDDEN - pos.shape[1]))
                 ).reshape(_NROWS, 128)

    def _kron8(W):
        a, b = W.shape[-2], W.shape[-1]
        lead = W.shape[:-2]
        out = (eye8[:, None, :, None] * W[..., None, :, None, :])
        return out.reshape(lead + (8 * a, 8 * b))

    for l, layer in enumerate(params["layers"]):
        d_in = 2 if l == 0 else _HIDDEN
        mlp = layer["edge_mlp"]
        Wf, bf = mlp[0]
        bf2 = bf.reshape(1, -1)
        Wm = jnp.stack([W for W, _ in mlp[1:15]])
        bm = jnp.stack([b.reshape(1, -1) for _, b in mlp[1:15]])
        Wl, bl = mlp[15]
        bl2 = bl.reshape(1, -1)
        rootW = layer["root_W"]
        if rootW.shape[0] < _HIDDEN:
            rootW = jnp.pad(rootW, ((0, _HIDDEN - rootW.shape[0]), (0, 0)))
        rWP = _kron8(rootW)
        biasP = jnp.tile(layer["bias"].reshape(1, -1), (1, 8))

        x16 = xP.reshape(_N_NODES, _HIDDEN)
        xs, xd = _sc_gather(x16, src2, dst2)
        xsP = xs.reshape(_N_EDGES // 8, 128)
        xdP = xd.reshape(_N_EDGES // 8, 128)
        msgP = _edge_mlp(eaP, xsP, xdP, Wf, bf2, Wm, bm, Wl, bl2, d_in)
        part = _sc_scatter(msgP.reshape(_N_EDGES, _HIDDEN), dst3, zeros)
        partP = part.reshape(2, _NROWS, 128)
        xP = _node_update(xP, partP, rWP, biasP, residual=(l > 0))

    out = _readout(xP.reshape(_N_NODES, _HIDDEN),
                   batch_index.astype(jnp.int32).reshape(_N_NODES, 1),
                   params["proj_W"], params["proj_b"].reshape(1, 1))
    return out.reshape(_N_GRAPHS)
